# baseline (speedup 1.0000x reference)
import functools

import jax
import jax.numpy as jnp
from jax.experimental import pallas as pl

N_SDR = 16384
N_COLS = 2048
W_SPARSE = 40
K_ACTIVE = 40
_PERIODS12 = [31.0, 31.0, 37.0, 37.0, 41.0, 41.0, 43.0, 43.0, 47.0, 47.0, 53.0, 53.0]

_MV_BLK = 1024


def _matvec_body(w_ref, s_ref, out_ref):
    out_ref[...] = jnp.dot(w_ref[...], s_ref[...],
                           preferred_element_type=jnp.float32)


def _topk_sdr_body(ov_ref, sdr_ref, idxw_ref):
    fi = (jax.lax.broadcasted_iota(jnp.int32, (128, 128), 0) * 128
          + jax.lax.broadcasted_iota(jnp.int32, (128, 128), 1))
    iota40 = jax.lax.broadcasted_iota(jnp.int32, (W_SPARSE, 1), 0)

    def body(i, carry):
        vals, sdr, idxw = carry
        m = jnp.max(vals)
        j = jnp.min(jnp.where(vals >= m, fi, jnp.int32(2 ** 30)))
        sdr = jnp.where(fi == j, jnp.float32(1.0), sdr)
        vals = jnp.where(fi == j, jnp.float32(-jnp.inf), vals)
        idxw = jnp.where(iota40 == i, j, idxw)
        return vals, sdr, idxw

    _, sdr, idxw = jax.lax.fori_loop(
        0, W_SPARSE,
        body,
        (ov_ref[...], jnp.zeros((128, 128), jnp.float32),
         jnp.zeros((W_SPARSE, 1), jnp.int32)))
    sdr_ref[...] = sdr
    idxw_ref[...] = idxw


def _pooler_body(p_ref, sdr_ref, out_ref):
    i = pl.program_id(0)
    conn = (p_ref[...] > 0.5).astype(jnp.float32)
    part = jnp.dot(conn, sdr_ref[...], preferred_element_type=jnp.float32)

    @pl.when(i == 0)
    def _():
        out_ref[...] = part

    @pl.when(i > 0)
    def _():
        out_ref[...] = out_ref[...] + part


def _tail_body(colov_ref, wm_ref, wt_ref, b_ref, v_ref, ph0_ref, per_ref,
               act_ref, allo_ref, phase_ref, gc_ref):
    fi = (jax.lax.broadcasted_iota(jnp.int32, (16, 128), 0) * 128
          + jax.lax.broadcasted_iota(jnp.int32, (16, 128), 1))
    iota40 = jax.lax.broadcasted_iota(jnp.int32, (K_ACTIVE, 1), 0)
    iota2048 = jax.lax.broadcasted_iota(jnp.int32, (N_COLS, 1), 0)
    packed0 = (colov_ref[...] * jnp.float32(N_COLS)
               + (jnp.float32(N_COLS - 1) - fi.astype(jnp.float32)))

    def body(i, carry):
        packed, act, ad = carry
        m = jnp.max(packed)
        v = jnp.floor(m / jnp.float32(N_COLS))
        idx = (jnp.float32(N_COLS - 1) - (m - v * jnp.float32(N_COLS))
               ).astype(jnp.int32)
        packed = jnp.where(fi == idx, jnp.float32(-1.0), packed)
        act = jnp.where(iota40 == i, idx, act)
        ad = jnp.where(iota2048 == idx, jnp.float32(1.0), ad)
        return packed, act, ad

    _, act, ad = jax.lax.fori_loop(
        0, K_ACTIVE,
        body,
        (packed0, jnp.zeros((K_ACTIVE, 1), jnp.int32),
         jnp.zeros((N_COLS, 1), jnp.float32)))
    act_ref[...] = act

    v0 = v_ref[0, 0]
    v1 = v_ref[1, 0]
    pre = (jnp.dot(wm_ref[...], ad, preferred_element_type=jnp.float32)
           + wt_ref[:, 0:1] * v0 + wt_ref[:, 1:2] * v1 + b_ref[...])
    allo = jnp.tanh(pre)
    allo_ref[...] = allo
    phase = jnp.mod(ph0_ref[...] + allo / per_ref[...], jnp.float32(1.0))
    phase_ref[...] = phase
    ang = jnp.float32(2.0 * jnp.pi) * phase
    gc_ref[...] = jnp.concatenate([jnp.cos(ang), jnp.sin(ang)], axis=0)


@jax.jit
def kernel(s_t, v_t, W_enc, permanences, W_l6b, b_l6b, phase0):
    f32 = jnp.float32
    s_col = s_t.reshape(4096, 1)

    overlaps = pl.pallas_call(
        _matvec_body,
        grid=(N_SDR // _MV_BLK,),
        in_specs=[
            pl.BlockSpec((_MV_BLK, 4096), lambda i: (i, 0)),
            pl.BlockSpec((4096, 1), lambda i: (0, 0)),
        ],
        out_specs=pl.BlockSpec((_MV_BLK, 1), lambda i: (i, 0)),
        out_shape=jax.ShapeDtypeStruct((N_SDR, 1), f32),
    )(W_enc, s_col)

    sdr128, idx_w = pl.pallas_call(
        _topk_sdr_body,
        in_specs=[pl.BlockSpec((128, 128), lambda: (0, 0))],
        out_specs=[pl.BlockSpec((128, 128), lambda: (0, 0)),
                   pl.BlockSpec((W_SPARSE, 1), lambda: (0, 0))],
        out_shape=[jax.ShapeDtypeStruct((128, 128), f32),
                   jax.ShapeDtypeStruct((W_SPARSE, 1), jnp.int32)],
    )(overlaps.reshape(128, 128))
    sdr = sdr128.reshape(N_SDR)
    del idx_w

    _SP_BLK = 2048
    colov = pl.pallas_call(
        _pooler_body,
        grid=(N_SDR // _SP_BLK,),
        in_specs=[
            pl.BlockSpec((N_COLS, _SP_BLK), lambda i: (0, i)),
            pl.BlockSpec((_SP_BLK, 1), lambda i: (i, 0)),
        ],
        out_specs=pl.BlockSpec((N_COLS, 1), lambda i: (0, 0)),
        out_shape=jax.ShapeDtypeStruct((N_COLS, 1), f32),
    )(permanences, sdr.reshape(N_SDR, 1))

    active2d, allo2d, phase2d, gc2d = pl.pallas_call(
        _tail_body,
        in_specs=[
            pl.BlockSpec((16, 128), lambda: (0, 0)),
            pl.BlockSpec((12, N_COLS), lambda: (0, 0)),
            pl.BlockSpec((12, 2), lambda: (0, 0)),
            pl.BlockSpec((12, 1), lambda: (0, 0)),
            pl.BlockSpec((2, 1), lambda: (0, 0)),
            pl.BlockSpec((12, 1), lambda: (0, 0)),
            pl.BlockSpec((12, 1), lambda: (0, 0)),
        ],
        out_specs=[pl.BlockSpec((K_ACTIVE, 1), lambda: (0, 0)),
                   pl.BlockSpec((12, 1), lambda: (0, 0)),
                   pl.BlockSpec((12, 1), lambda: (0, 0)),
                   pl.BlockSpec((24, 1), lambda: (0, 0))],
        out_shape=[jax.ShapeDtypeStruct((K_ACTIVE, 1), jnp.int32),
                   jax.ShapeDtypeStruct((12, 1), f32),
                   jax.ShapeDtypeStruct((12, 1), f32),
                   jax.ShapeDtypeStruct((24, 1), f32)],
    )(colov.reshape(16, 128), W_l6b[:, :N_COLS], W_l6b[:, N_SDR:],
      b_l6b.reshape(12, 1), v_t.reshape(2, 1), phase0.reshape(12, 1),
      jnp.asarray(_PERIODS12, dtype=f32).reshape(12, 1))

    return (sdr, active2d.reshape(K_ACTIVE), allo2d.reshape(12),
            phase2d.reshape(6, 2), gc2d.reshape(24))
